# Initial kernel scaffold; baseline (speedup 1.0000x reference)
import functools

import jax
import jax.numpy as jnp
from jax import lax
from jax.experimental import pallas as pl
from jax.experimental.pallas import tpu as pltpu
from jax.experimental.pallas import tpu_sc as plsc

N_NODES = 10000
N_EDGES = 320000
D_FEAT = 128

NC = 2
NS = 16
L = 16
NW = NC * NS
E_PER_W = N_EDGES // NW
N_PAD = 10240

NW_B = 25
NODES_B = N_NODES // NW_B
OUT_PER_W = NODES_B * D_FEAT

_mesh = plsc.VectorSubcoreMesh(
    core_axis_name="c", subcore_axis_name="s", num_cores=NC, num_subcores=NS
)


@functools.partial(
    pl.kernel,
    out_type=jax.ShapeDtypeStruct((NW, N_PAD), jnp.float32),
    mesh=_mesh,
    scratch_types=[
        pltpu.VMEM((E_PER_W,), jnp.int32),
        pltpu.VMEM((N_PAD,), jnp.float32),
        pltpu.SemaphoreType.DMA,
    ],
)
def _scatter_flags(idx_hbm, part_hbm, idx_v, flags_v, sem):
    wid = lax.axis_index("s") * NC + lax.axis_index("c")
    base = wid * E_PER_W
    cp = pltpu.async_copy(idx_hbm.at[pl.ds(base, E_PER_W)], idx_v, sem)

    zero = jnp.zeros((L,), jnp.float32)

    def zbody(i, carry):
        flags_v[pl.ds(i * L, L)] = zero
        return carry

    lax.fori_loop(0, N_PAD // L, zbody, 0)
    cp.wait()

    one = jnp.ones((L,), jnp.float32)

    def sbody(i, carry):
        iv = idx_v[pl.ds(i * L, L)]
        plsc.store_scatter(flags_v, [iv], one)
        return carry

    lax.fori_loop(0, E_PER_W // L, sbody, 0)
    pltpu.sync_copy(flags_v, part_hbm.at[wid])


@functools.partial(
    pl.kernel,
    out_type=jax.ShapeDtypeStruct((N_NODES * D_FEAT,), jnp.float32),
    mesh=_mesh,
    scratch_types=[
        pltpu.VMEM((NW, NODES_B), jnp.float32),
        pltpu.VMEM((NODES_B,), jnp.float32),
        pltpu.VMEM((OUT_PER_W,), jnp.float32),
    ],
)
def _reduce_broadcast(part_hbm, out_hbm, pblk_v, flags_v, out_v):
    wid = lax.axis_index("s") * NC + lax.axis_index("c")

    @pl.when(wid < NW_B)
    def _():
        nbase = wid * NODES_B
        pltpu.sync_copy(part_hbm.at[:, pl.ds(nbase, NODES_B)], pblk_v)

        for g in range(NODES_B // L):
            acc = pblk_v[0, pl.ds(g * L, L)]
            for r in range(1, NW):
                acc = acc + pblk_v[r, pl.ds(g * L, L)]
            flags_v[pl.ds(g * L, L)] = jnp.where(acc > 0.0, 1.0, 0.0)

        def bbody(n, carry):
            iv = jnp.full((L,), n, dtype=jnp.int32)
            fv = plsc.load_gather(flags_v, [iv])
            for k in range(D_FEAT // L):
                out_v[pl.ds(n * D_FEAT + k * L, L)] = fv
            return carry

        lax.fori_loop(0, NODES_B, bbody, 0)
        pltpu.sync_copy(out_v, out_hbm.at[pl.ds(nbase * D_FEAT, OUT_PER_W)])


def kernel(source_node_representation_with_coefficient, edge_index):
    del source_node_representation_with_coefficient
    idx = edge_index[1]
    part = _scatter_flags(idx)
    out_flat = _reduce_broadcast(part)
    return out_flat.reshape(N_NODES, D_FEAT)


# speedup vs baseline: 77.7771x; 77.7771x over previous
import functools

import jax
import jax.numpy as jnp
from jax import lax
from jax.experimental import pallas as pl
from jax.experimental.pallas import tpu as pltpu
from jax.experimental.pallas import tpu_sc as plsc

N_NODES = 10000
N_EDGES = 320000
D_FEAT = 128

NC = 2
NS = 16
L = 16
NW = NC * NS
E_PER_W = N_EDGES // NW
N_PAD = 10240

NW_B = 25
NODES_B = N_NODES // NW_B
OUT_PER_W = NODES_B * D_FEAT

_mesh = plsc.VectorSubcoreMesh(
    core_axis_name="c", subcore_axis_name="s", num_cores=NC, num_subcores=NS
)

_params = pltpu.CompilerParams(
    needs_layout_passes=False, use_tc_tiling_on_sc=False
)


@functools.partial(
    pl.kernel,
    out_type=jax.ShapeDtypeStruct((NW, N_PAD), jnp.float32),
    mesh=_mesh,
    scratch_types=[
        pltpu.VMEM((E_PER_W,), jnp.int32),
        pltpu.VMEM((N_PAD,), jnp.float32),
        pltpu.SemaphoreType.DMA,
    ],
    compiler_params=_params,
)
def _scatter_flags(idx_hbm, part_hbm, idx_v, flags_v, sem):
    wid = lax.axis_index("s") * NC + lax.axis_index("c")
    base = wid * E_PER_W
    cp = pltpu.async_copy(idx_hbm.at[pl.ds(base, E_PER_W)], idx_v, sem)

    zero = jnp.zeros((L,), jnp.float32)

    def zbody(i, carry):
        flags_v[pl.ds(i * L, L)] = zero
        return carry

    lax.fori_loop(0, N_PAD // L, zbody, 0)
    cp.wait()

    one = jnp.ones((L,), jnp.float32)

    def sbody(i, carry):
        iv = idx_v[pl.ds(i * L, L)]
        plsc.store_scatter(flags_v, [iv], one)
        return carry

    lax.fori_loop(0, E_PER_W // L, sbody, 0)
    pltpu.sync_copy(flags_v, part_hbm.at[wid])


@functools.partial(
    pl.kernel,
    out_type=jax.ShapeDtypeStruct((N_NODES * D_FEAT,), jnp.float32),
    mesh=_mesh,
    scratch_types=[
        pltpu.VMEM((NW, NODES_B), jnp.float32),
        pltpu.VMEM((NODES_B,), jnp.float32),
        pltpu.VMEM((OUT_PER_W,), jnp.float32),
    ],
    compiler_params=_params,
)
def _reduce_broadcast(part_hbm, out_hbm, pblk_v, flags_v, out_v):
    wid = lax.axis_index("s") * NC + lax.axis_index("c")

    @pl.when(wid < NW_B)
    def _():
        nbase = wid * NODES_B
        pltpu.sync_copy(part_hbm.at[:, pl.ds(nbase, NODES_B)], pblk_v)

        for g in range(NODES_B // L):
            acc = pblk_v[0, pl.ds(g * L, L)]
            for r in range(1, NW):
                acc = acc + pblk_v[r, pl.ds(g * L, L)]
            flags_v[pl.ds(g * L, L)] = jnp.where(acc > 0.0, 1.0, 0.0)

        def bbody(n, carry):
            iv = jnp.full((L,), n, dtype=jnp.int32)
            fv = plsc.load_gather(flags_v, [iv])
            for k in range(D_FEAT // L):
                out_v[pl.ds(n * D_FEAT + k * L, L)] = fv
            return carry

        lax.fori_loop(0, NODES_B, bbody, 0)
        pltpu.sync_copy(out_v, out_hbm.at[pl.ds(nbase * D_FEAT, OUT_PER_W)])


def kernel(source_node_representation_with_coefficient, edge_index):
    del source_node_representation_with_coefficient
    idx = edge_index[1]
    part = _scatter_flags(idx)
    out_flat = _reduce_broadcast(part)
    return out_flat.reshape(N_NODES, D_FEAT)
